# baseline (speedup 1.0000x reference)
import functools

import jax
import jax.numpy as jnp
from jax import lax
from jax.experimental import pallas as pl
from jax.experimental.pallas import tpu as pltpu
from jax.experimental.pallas import tpu_sc as plsc

N = 10000
E = 320000
D = 128
G = 128

NC = 2
NS = 16
NW = NC * NS
EPW = E // NW
ECORE = E // NC
CH = 40
NB = 5
LAG = 2
PH = 5
CPP = 50
NITER = CPP // NB
RPT = 624
RTAIL = N - NS * RPT

BN = 1000
GRID = N // BN

_SC_MESH = plsc.VectorSubcoreMesh(core_axis_name="c", subcore_axis_name="s")


def _deg_body(dst_hbm, out_hbm, dloc, degloc):
    c = lax.axis_index("c")
    s = lax.axis_index("s")
    wid = c * NS + s
    zero16 = jnp.zeros((16,), jnp.float32)

    def zbody(k, _):
        degloc[pl.ds(k * 16, 16)] = zero16
        return 0

    lax.fori_loop(0, N // 16, zbody, 0)
    ebase = pl.multiple_of(wid * EPW, 8)
    pltpu.sync_copy(dst_hbm.at[pl.ds(ebase, EPW)], dloc)
    one16 = jnp.ones((16,), jnp.float32)

    def sbody(k, _):
        idx = dloc[pl.ds(k * 16, 16)]
        plsc.addupdate_scatter(degloc, [idx], one16)
        return 0

    lax.fori_loop(0, EPW // 16, sbody, 0)
    obase = pl.multiple_of(wid * N, 8)
    pltpu.sync_copy(degloc, out_hbm.at[pl.ds(obase, N)])


_deg_call = functools.partial(
    pl.kernel,
    out_type=jax.ShapeDtypeStruct((NW * N,), jnp.float32),
    mesh=_SC_MESH,
    compiler_params=pltpu.CompilerParams(needs_layout_passes=False),
    scratch_types=[
        pltpu.VMEM((EPW,), jnp.int32),
        pltpu.VMEM((N,), jnp.float32),
    ],
)(_deg_body)


def _edge_body(src4_hbm, dst4_hbm, hp_hbm, zeros_hbm, out_hbm,
               acc, sloc, dloc, r0, r1, r2, r3, r4,
               isem, gs0, gs1, gs2, gs3, gs4, ss0, ss1, ss2, ss3, ss4):
    c = lax.axis_index("c")
    s = lax.axis_index("s")
    wid = c * NS + s
    pltpu.sync_copy(zeros_hbm.at[pl.ds(s * RPT, RPT)],
                    acc.at[pl.ds(s * RPT, RPT)])

    @pl.when(s == 0)
    def _():
        pltpu.sync_copy(zeros_hbm.at[pl.ds(NS * RPT, RTAIL)],
                        acc.at[pl.ds(NS * RPT, RTAIL)])

    plsc.subcore_barrier()

    rows = [r0, r1, r2, r3, r4]
    gsems = [gs0, gs1, gs2, gs3, gs4]
    ssems = [ss0, ss1, ss2, ss3, ss4]

    for p in range(PH):
        di1 = pltpu.async_copy(src4_hbm.at[wid, p], sloc, isem)
        di2 = pltpu.async_copy(dst4_hbm.at[wid, p], dloc, isem)
        di1.wait()
        di2.wait()
        for k in range(NB):
            pltpu.async_copy(hp_hbm.at[sloc.at[k]], rows[k], gsems[k])

        def body(j, _):
            base = j * NB
            for k in range(NB):
                ch = base + k
                pltpu.make_async_copy(hp_hbm.at[sloc.at[ch]], rows[k],
                                      gsems[k]).wait()
                pltpu.async_copy(rows[k], acc.at[dloc.at[ch]], ssems[k],
                                 add=True)
                km = (k - LAG) % NB
                tm = ch - LAG

                @pl.when((tm >= 0) & (tm + NB < CPP))
                def _():
                    pltpu.make_async_copy(rows[km], acc.at[dloc.at[0]],
                                          ssems[km]).wait()
                    pltpu.async_copy(hp_hbm.at[sloc.at[tm + NB]], rows[km],
                                     gsems[km])

            return 0

        lax.fori_loop(0, NITER, body, 0)
        for k in range(NB):
            pltpu.make_async_copy(rows[k], acc.at[dloc.at[0]],
                                  ssems[k]).wait()

    plsc.subcore_barrier()
    pltpu.sync_copy(acc.at[pl.ds(s * RPT, RPT)],
                    out_hbm.at[pl.ds(c * N + s * RPT, RPT)])

    @pl.when(s == 0)
    def _():
        pltpu.sync_copy(acc.at[pl.ds(NS * RPT, RTAIL)],
                        out_hbm.at[pl.ds(c * N + NS * RPT, RTAIL)])


_edge_call = functools.partial(
    pl.kernel,
    out_type=jax.ShapeDtypeStruct((NC * N, D), jnp.float32),
    mesh=_SC_MESH,
    scratch_types=[
        pltpu.VMEM_SHARED((N, D), jnp.float32),
        pltpu.VMEM((CPP, CH), jnp.int32),
        pltpu.VMEM((CPP, CH), jnp.int32),
        pltpu.VMEM((CH, D), jnp.float32),
        pltpu.VMEM((CH, D), jnp.float32),
        pltpu.VMEM((CH, D), jnp.float32),
        pltpu.VMEM((CH, D), jnp.float32),
        pltpu.VMEM((CH, D), jnp.float32),
        pltpu.SemaphoreType.DMA,
        pltpu.SemaphoreType.DMA,
        pltpu.SemaphoreType.DMA,
        pltpu.SemaphoreType.DMA,
        pltpu.SemaphoreType.DMA,
        pltpu.SemaphoreType.DMA,
        pltpu.SemaphoreType.DMA,
        pltpu.SemaphoreType.DMA,
        pltpu.SemaphoreType.DMA,
        pltpu.SemaphoreType.DMA,
        pltpu.SemaphoreType.DMA,
    ],
)(_edge_body)


def _pre_body(x_ref, degp_ref, wexp_ref, bexp_ref, w0_ref,
              atoms_ref, dis_ref, hp_ref):
    deg = jnp.sum(degp_ref[...], axis=0) + 1.0
    dis = lax.rsqrt(deg)
    atoms = jnp.log(x_ref[...] + 1.0) @ wexp_ref[...] + bexp_ref[...]
    atoms_ref[...] = atoms
    dis_ref[...] = dis
    hp_ref[...] = (atoms @ w0_ref[...]) * dis


def _finalize(p0, p1, hp, dis, b, g, be, atoms):
    agg = (p0 + p1 + hp) * dis + b
    mean = jnp.mean(agg, axis=-1, keepdims=True)
    var = jnp.mean((agg - mean) ** 2, axis=-1, keepdims=True)
    h = (agg - mean) * lax.rsqrt(var + 1e-5) * g + be
    h = 0.5 * h * (1.0 + lax.erf(h * 0.7071067811865475))
    return atoms + h


def _layer_body(p0_ref, p1_ref, hp_ref, dis_ref, b_ref, g_ref, be_ref,
                atoms_ref, wn_ref, atomsn_ref, hpn_ref):
    dis = dis_ref[...]
    atoms_n = _finalize(p0_ref[...], p1_ref[...], hp_ref[...], dis,
                        b_ref[...], g_ref[...], be_ref[...], atoms_ref[...])
    atomsn_ref[...] = atoms_n
    hpn_ref[...] = (atoms_n @ wn_ref[...]) * dis


def _final_body(p0_ref, p1_ref, hp_ref, dis_ref, b_ref, g_ref, be_ref,
                atoms_ref, batch_ref, out_ref):
    atoms_n = _finalize(p0_ref[...], p1_ref[...], hp_ref[...], dis_ref[...],
                        b_ref[...], g_ref[...], be_ref[...], atoms_ref[...])
    oh = (batch_ref[...] == lax.broadcasted_iota(jnp.int32, (BN, G), 1))
    contrib = lax.dot_general(oh.astype(jnp.float32), atoms_n,
                              (((0,), (0,)), ((), ())),
                              preferred_element_type=jnp.float32)

    @pl.when(pl.program_id(0) == 0)
    def _():
        out_ref[...] = jnp.zeros_like(out_ref)

    out_ref[...] += contrib


def _row_spec(i):
    del i
    return pl.BlockSpec((BN, D), lambda i: (i, 0))


_ROW = pl.BlockSpec((BN, D), lambda i: (i, 0))
_ROW1 = pl.BlockSpec((BN, 1), lambda i: (i, 0))
_FULL_W = pl.BlockSpec((D, D), lambda i: (0, 0))
_FULL_V = pl.BlockSpec((D,), lambda i: (0,))

_pre_call = pl.pallas_call(
    _pre_body,
    grid=(GRID,),
    in_specs=[
        pl.BlockSpec((BN, 8), lambda i: (i, 0)),
        pl.BlockSpec((NW, BN, 1), lambda i: (0, i, 0)),
        pl.BlockSpec((8, D), lambda i: (0, 0)),
        _FULL_V,
        _FULL_W,
    ],
    out_specs=[_ROW, _ROW1, _ROW],
    out_shape=[
        jax.ShapeDtypeStruct((N, D), jnp.float32),
        jax.ShapeDtypeStruct((N, 1), jnp.float32),
        jax.ShapeDtypeStruct((N, D), jnp.float32),
    ],
)

_P0 = pl.BlockSpec((BN, D), lambda i: (i, 0))
_P1 = pl.BlockSpec((BN, D), lambda i: (i + GRID, 0))

_layer_call = pl.pallas_call(
    _layer_body,
    grid=(GRID,),
    in_specs=[_P0, _P1, _ROW, _ROW1, _FULL_V, _FULL_V, _FULL_V, _ROW, _FULL_W],
    out_specs=[_ROW, _ROW],
    out_shape=[
        jax.ShapeDtypeStruct((N, D), jnp.float32),
        jax.ShapeDtypeStruct((N, D), jnp.float32),
    ],
)

_final_call = pl.pallas_call(
    _final_body,
    grid=(GRID,),
    in_specs=[_P0, _P1, _ROW, _ROW1, _FULL_V, _FULL_V, _FULL_V, _ROW,
              pl.BlockSpec((BN, 1), lambda i: (i, 0))],
    out_specs=pl.BlockSpec((G, D), lambda i: (0, 0)),
    out_shape=jax.ShapeDtypeStruct((G, D), jnp.float32),
)


def kernel(x, edge_index, batch, Wexp, bexp,
           W0, b0, g0, be0, W1, b1, g1, be1, W2, b2, g2, be2):
    src = edge_index[0]
    dst = edge_index[1]
    src4 = src.reshape(NW, PH, CPP, CH)
    dst4 = dst.reshape(NW, PH, CPP, CH)
    zeros2d = jnp.zeros((N, D), jnp.float32)

    degp = _deg_call(dst).reshape(NW, N, 1)
    atoms, dis, hp = _pre_call(x, degp, Wexp, bexp, W0)

    params = [(b0, g0, be0, W1), (b1, g1, be1, W2), (b2, g2, be2, None)]
    for li, (b, g, be, wn) in enumerate(params):
        part = _edge_call(src4, dst4, hp, zeros2d)
        if wn is None:
            return _final_call(part, part, hp, dis, b, g, be, atoms,
                               batch.reshape(N, 1))
        atoms, hp = _layer_call(part, part, hp, dis, b, g, be, atoms, wn)


# speedup vs baseline: 1.5386x; 1.5386x over previous
import functools

import jax
import jax.numpy as jnp
from jax import lax
from jax.experimental import pallas as pl
from jax.experimental.pallas import tpu as pltpu
from jax.experimental.pallas import tpu_sc as plsc

N = 10000
E = 320000
D = 128
G = 128

NC = 2
NS = 16
NW = NC * NS
EPW = E // NW
CH = 40
NB = 5
PH = 5
CPP = 50
NITER = CPP // NB
RPT = 624
RTAIL = N - NS * RPT

BN = 1000
GRID = N // BN

_SC_MESH = plsc.VectorSubcoreMesh(core_axis_name="c", subcore_axis_name="s")


NPR = 80


def _deg_body(dst_hbm, zeros_hbm, rid_hbm, out_hbm, dloc, degloc, rid, acc):
    c = lax.axis_index("c")
    s = lax.axis_index("s")
    wid = c * NS + s
    pltpu.sync_copy(zeros_hbm.at[pl.ds(0, NPR)], degloc)
    pltpu.sync_copy(rid_hbm, rid)

    @pl.when(s == 0)
    def _():
        pltpu.sync_copy(zeros_hbm.at[pl.ds(0, NPR)], acc)

    ebase = pl.multiple_of(wid * EPW, 8)
    pltpu.sync_copy(dst_hbm.at[pl.ds(ebase, EPW)], dloc)
    one16 = jnp.ones((16,), jnp.float32)

    def sbody(k, _):
        idx = dloc[pl.ds(k * 16, 16)]
        plsc.addupdate_scatter(degloc, [idx >> 7, idx & 127], one16)
        return 0

    lax.fori_loop(0, EPW // 16, sbody, 0)
    plsc.subcore_barrier()
    pltpu.sync_copy(degloc, acc.at[rid], add=True)
    plsc.subcore_barrier()

    @pl.when(s == 0)
    def _():
        pltpu.sync_copy(acc, out_hbm.at[c])


_deg_call = functools.partial(
    pl.kernel,
    out_type=jax.ShapeDtypeStruct((NC, NPR, 128), jnp.float32),
    mesh=_SC_MESH,
    compiler_params=pltpu.CompilerParams(needs_layout_passes=False),
    scratch_types=[
        pltpu.VMEM((EPW,), jnp.int32),
        pltpu.VMEM((NPR, 128), jnp.float32),
        pltpu.VMEM((NPR,), jnp.int32),
        pltpu.VMEM_SHARED((NPR, 128), jnp.float32),
    ],
)(_deg_body)


def _edge_body(src4_hbm, dst4_hbm, hp_hbm, zeros_hbm, out_hbm,
               acc, sloc, dloc, r0, r1, r2, r3, r4,
               isem, gs0, gs1, gs2, gs3, gs4, ss0, ss1, ss2, ss3, ss4):
    c = lax.axis_index("c")
    s = lax.axis_index("s")
    wid = c * NS + s
    pltpu.sync_copy(zeros_hbm.at[pl.ds(s * RPT, RPT)],
                    acc.at[pl.ds(s * RPT, RPT)])

    @pl.when(s == 0)
    def _():
        pltpu.sync_copy(zeros_hbm.at[pl.ds(NS * RPT, RTAIL)],
                        acc.at[pl.ds(NS * RPT, RTAIL)])

    plsc.subcore_barrier()

    rows = [r0, r1, r2, r3, r4]
    gsems = [gs0, gs1, gs2, gs3, gs4]
    ssems = [ss0, ss1, ss2, ss3, ss4]

    for p in range(PH):
        di1 = pltpu.async_copy(src4_hbm.at[wid, p], sloc, isem)
        di2 = pltpu.async_copy(dst4_hbm.at[wid, p], dloc, isem)
        di1.wait()
        di2.wait()
        for k in range(NB):
            pltpu.async_copy(hp_hbm.at[sloc.at[k]], rows[k], gsems[k])

        def body(j, _):
            base = j * NB
            for k in range(NB):
                ch = base + k
                pltpu.make_async_copy(hp_hbm.at[sloc.at[ch]], rows[k],
                                      gsems[k]).wait()
                pltpu.async_copy(rows[k], acc.at[dloc.at[ch]], ssems[k],
                                 add=True).wait()

                @pl.when(j < NITER - 1)
                def _():
                    pltpu.async_copy(hp_hbm.at[sloc.at[ch + NB]], rows[k],
                                     gsems[k])

            return 0

        lax.fori_loop(0, NITER, body, 0)

    plsc.subcore_barrier()
    pltpu.sync_copy(acc.at[pl.ds(s * RPT, RPT)],
                    out_hbm.at[pl.ds(c * N + s * RPT, RPT)])

    @pl.when(s == 0)
    def _():
        pltpu.sync_copy(acc.at[pl.ds(NS * RPT, RTAIL)],
                        out_hbm.at[pl.ds(c * N + NS * RPT, RTAIL)])


_edge_call = functools.partial(
    pl.kernel,
    out_type=jax.ShapeDtypeStruct((NC * N, D), jnp.float32),
    mesh=_SC_MESH,
    scratch_types=[
        pltpu.VMEM_SHARED((N, D), jnp.float32),
        pltpu.VMEM((CPP, CH), jnp.int32),
        pltpu.VMEM((CPP, CH), jnp.int32),
        pltpu.VMEM((CH, D), jnp.float32),
        pltpu.VMEM((CH, D), jnp.float32),
        pltpu.VMEM((CH, D), jnp.float32),
        pltpu.VMEM((CH, D), jnp.float32),
        pltpu.VMEM((CH, D), jnp.float32),
        pltpu.SemaphoreType.DMA,
        pltpu.SemaphoreType.DMA,
        pltpu.SemaphoreType.DMA,
        pltpu.SemaphoreType.DMA,
        pltpu.SemaphoreType.DMA,
        pltpu.SemaphoreType.DMA,
        pltpu.SemaphoreType.DMA,
        pltpu.SemaphoreType.DMA,
        pltpu.SemaphoreType.DMA,
        pltpu.SemaphoreType.DMA,
        pltpu.SemaphoreType.DMA,
    ],
)(_edge_body)


def _pre_body(x_ref, deg_ref, wexp_ref, bexp_ref, w0_ref,
              atoms_ref, dis_ref, hp_ref):
    dis = lax.rsqrt(deg_ref[...] + 1.0)
    atoms = jnp.log(x_ref[...] + 1.0) @ wexp_ref[...] + bexp_ref[...]
    atoms_ref[...] = atoms
    dis_ref[...] = dis
    hp_ref[...] = (atoms @ w0_ref[...]) * dis


def _finalize(p0, p1, hp, dis, b, g, be, atoms):
    agg = (p0 + p1 + hp) * dis + b
    mean = jnp.mean(agg, axis=-1, keepdims=True)
    var = jnp.mean((agg - mean) ** 2, axis=-1, keepdims=True)
    h = (agg - mean) * lax.rsqrt(var + 1e-5) * g + be
    h = 0.5 * h * (1.0 + lax.erf(h * 0.7071067811865475))
    return atoms + h


def _layer_body(p0_ref, p1_ref, hp_ref, dis_ref, b_ref, g_ref, be_ref,
                atoms_ref, wn_ref, atomsn_ref, hpn_ref):
    dis = dis_ref[...]
    atoms_n = _finalize(p0_ref[...], p1_ref[...], hp_ref[...], dis,
                        b_ref[...], g_ref[...], be_ref[...], atoms_ref[...])
    atomsn_ref[...] = atoms_n
    hpn_ref[...] = (atoms_n @ wn_ref[...]) * dis


def _final_body(p0_ref, p1_ref, hp_ref, dis_ref, b_ref, g_ref, be_ref,
                atoms_ref, batch_ref, out_ref):
    atoms_n = _finalize(p0_ref[...], p1_ref[...], hp_ref[...], dis_ref[...],
                        b_ref[...], g_ref[...], be_ref[...], atoms_ref[...])
    oh = (batch_ref[...] == lax.broadcasted_iota(jnp.int32, (BN, G), 1))
    contrib = lax.dot_general(oh.astype(jnp.float32), atoms_n,
                              (((0,), (0,)), ((), ())),
                              preferred_element_type=jnp.float32)

    @pl.when(pl.program_id(0) == 0)
    def _():
        out_ref[...] = jnp.zeros_like(out_ref)

    out_ref[...] += contrib


_ROW = pl.BlockSpec((BN, D), lambda i: (i, 0))
_ROW1 = pl.BlockSpec((BN, 1), lambda i: (i, 0))
_FULL_W = pl.BlockSpec((D, D), lambda i: (0, 0))
_FULL_V = pl.BlockSpec((D,), lambda i: (0,))
_P0 = pl.BlockSpec((BN, D), lambda i: (i, 0))
_P1 = pl.BlockSpec((BN, D), lambda i: (i + GRID, 0))

_pre_call = pl.pallas_call(
    _pre_body,
    grid=(GRID,),
    in_specs=[
        pl.BlockSpec((BN, 8), lambda i: (i, 0)),
        _ROW1,
        pl.BlockSpec((8, D), lambda i: (0, 0)),
        _FULL_V,
        _FULL_W,
    ],
    out_specs=[_ROW, _ROW1, _ROW],
    out_shape=[
        jax.ShapeDtypeStruct((N, D), jnp.float32),
        jax.ShapeDtypeStruct((N, 1), jnp.float32),
        jax.ShapeDtypeStruct((N, D), jnp.float32),
    ],
)

_layer_call = pl.pallas_call(
    _layer_body,
    grid=(GRID,),
    in_specs=[_P0, _P1, _ROW, _ROW1, _FULL_V, _FULL_V, _FULL_V, _ROW, _FULL_W],
    out_specs=[_ROW, _ROW],
    out_shape=[
        jax.ShapeDtypeStruct((N, D), jnp.float32),
        jax.ShapeDtypeStruct((N, D), jnp.float32),
    ],
)

_final_call = pl.pallas_call(
    _final_body,
    grid=(GRID,),
    in_specs=[_P0, _P1, _ROW, _ROW1, _FULL_V, _FULL_V, _FULL_V, _ROW,
              pl.BlockSpec((BN, 1), lambda i: (i, 0))],
    out_specs=pl.BlockSpec((G, D), lambda i: (0, 0)),
    out_shape=jax.ShapeDtypeStruct((G, D), jnp.float32),
)


def kernel(x, edge_index, batch, Wexp, bexp,
           W0, b0, g0, be0, W1, b1, g1, be1, W2, b2, g2, be2):
    src4 = edge_index[0].reshape(NW, PH, CPP, CH)
    dst4 = edge_index[1].reshape(NW, PH, CPP, CH)
    zeros2d = jnp.zeros((N, D), jnp.float32)
    rid = jnp.arange(NPR, dtype=jnp.int32)

    degp = _deg_call(edge_index[1], zeros2d, rid)
    deg1 = (degp[0] + degp[1]).reshape(NPR * 128)[:N].reshape(N, 1)
    atoms, dis, hp = _pre_call(x, deg1, Wexp, bexp, W0)

    params = [(b0, g0, be0, W1), (b1, g1, be1, W2), (b2, g2, be2, None)]
    for b, g, be, wn in params:
        part = _edge_call(src4, dst4, hp, zeros2d)
        if wn is None:
            return _final_call(part, part, hp, dis, b, g, be, atoms,
                               batch.reshape(N, 1))
        atoms, hp = _layer_call(part, part, hp, dis, b, g, be, atoms, wn)
